# baseline (speedup 1.0000x reference)
import functools

import jax
import jax.numpy as jnp
from jax import lax
from jax.experimental import pallas as pl
from jax.experimental.pallas import tpu as pltpu
from jax.experimental.pallas import tpu_sc as plsc

N_NODES = 10000
N_EDGES = 320000
F_IN = 128
HID = 10
FP = 16

NW = 32
CHUNK = 128
CPT = 80
EPT = CPT * CHUNK
EPAD = NW * EPT
NPAD = 10112
GPT = EPT // 16
STRIPE = NPAD // 16
NBUF = 8

@functools.cache
def _sc_mesh():
    return plsc.VectorSubcoreMesh(
        core_axis_name="c", subcore_axis_name="s", num_cores=2, num_subcores=16)


def _deg_body(row_hbm, zb1_hbm, deg_out, row_v, deg_v):
    cid = lax.axis_index("c")
    sid = lax.axis_index("s")
    wid = sid * 2 + cid
    pltpu.sync_copy(row_hbm.at[wid], row_v)
    pltpu.sync_copy(zb1_hbm, deg_v)
    ones16 = jnp.ones((16,), jnp.float32)

    def add_body(j, carry):
        plsc.addupdate_scatter(deg_v, [row_v[j]], ones16)
        return carry

    lax.fori_loop(0, GPT, add_body, 0, unroll=8)
    pltpu.sync_copy(deg_v, deg_out.at[wid])


@functools.cache
def _deg_kernel():
    return pl.kernel(
        _deg_body,
        out_type=jax.ShapeDtypeStruct((NW, NPAD), jnp.float32),
        mesh=_sc_mesh(),
        scratch_types=[
            pltpu.VMEM((GPT, 16), jnp.int32),
            pltpu.VMEM((NPAD,), jnp.float32),
        ],
        compiler_params=pltpu.CompilerParams(needs_layout_passes=False),
    )


def _msg_body(row_hbm, col_hbm, z_hbm, zb_hbm, acc_out,
              row_v, col_v, *rest):
    bufs = rest[:NBUF]
    acc_s = rest[NBUF]
    gsems = rest[NBUF + 1:2 * NBUF + 1]
    ssems = rest[2 * NBUF + 1:]
    cid = lax.axis_index("c")
    sid = lax.axis_index("s")
    wid = sid * 2 + cid
    pltpu.sync_copy(row_hbm.at[wid], row_v)
    pltpu.sync_copy(col_hbm.at[wid], col_v)
    pltpu.sync_copy(zb_hbm, acc_s.at[pl.ds(sid * STRIPE, STRIPE)])
    plsc.subcore_barrier()

    for i in range(NBUF):
        pltpu.async_copy(z_hbm.at[row_v.at[i]], bufs[i], gsems[i])

    def body(k, carry):
        j0 = NBUF * k
        for i in range(NBUF):
            pltpu.make_async_copy(
                z_hbm.at[row_v.at[j0 + i]], bufs[i], gsems[i]).wait()
            pltpu.async_copy(
                bufs[i], acc_s.at[col_v.at[j0 + i]], ssems[i], add=True)
        for i in range(NBUF):
            pltpu.make_async_copy(
                bufs[i], acc_s.at[col_v.at[j0 + i]], ssems[i]).wait()
            pltpu.async_copy(
                z_hbm.at[row_v.at[j0 + NBUF + i]], bufs[i], gsems[i])
        return carry

    lax.fori_loop(0, CPT // NBUF - 1, body, 0)
    j0 = (CPT // NBUF - 1) * NBUF
    for i in range(NBUF):
        pltpu.make_async_copy(
            z_hbm.at[row_v.at[j0 + i]], bufs[i], gsems[i]).wait()
        pltpu.async_copy(
            bufs[i], acc_s.at[col_v.at[j0 + i]], ssems[i], add=True)
    for i in range(NBUF):
        pltpu.make_async_copy(
            bufs[i], acc_s.at[col_v.at[j0 + i]], ssems[i]).wait()
    for j in range(NBUF * (CPT // NBUF), CPT):
        i = j % NBUF
        pltpu.sync_copy(z_hbm.at[row_v.at[j]], bufs[i])
        pltpu.sync_copy(bufs[i], acc_s.at[col_v.at[j]], add=True)
    plsc.subcore_barrier()
    pltpu.sync_copy(acc_s.at[pl.ds(sid * STRIPE, STRIPE)],
                    acc_out.at[cid, pl.ds(sid * STRIPE, STRIPE)])


@functools.cache
def _msg_kernel():
    return pl.kernel(
        _msg_body,
        out_type=jax.ShapeDtypeStruct((2, NPAD, FP), jnp.float32),
        mesh=_sc_mesh(),
        scratch_types=[
            pltpu.VMEM((CPT, CHUNK), jnp.int32),
            pltpu.VMEM((CPT, CHUNK), jnp.int32),
        ] + [pltpu.VMEM((CHUNK, FP), jnp.float32)] * NBUF + [
            pltpu.VMEM_SHARED((NPAD, FP), jnp.float32),
        ] + [pltpu.SemaphoreType.DMA] * (2 * NBUF),
        compiler_params=pltpu.CompilerParams(
            needs_layout_passes=False, use_tc_tiling_on_sc=False),
    )


def _tc_a_body(xp_ref, w0_ref, w1_ref, degs_ref, y0_ref, z_ref, dis_ref):
    deg = jnp.sum(degs_ref[...], axis=0)
    dis = jnp.where(deg > 0, lax.rsqrt(deg), 0.0)
    x = xp_ref[...]
    y0_ref[...] = jnp.dot(x, w0_ref[...], preferred_element_type=jnp.float32)
    y1 = jnp.dot(x, w1_ref[...], preferred_element_type=jnp.float32)
    z_ref[...] = y1 * dis[:, None]
    dis_ref[...] = dis


def _tc_a(xp, w0p, w1p, degs):
    return pl.pallas_call(
        _tc_a_body,
        out_shape=(
            jax.ShapeDtypeStruct((NPAD, FP), jnp.float32),
            jax.ShapeDtypeStruct((NPAD, FP), jnp.float32),
            jax.ShapeDtypeStruct((NPAD,), jnp.float32),
        ),
    )(xp, w0p, w1p, degs)


def _tc_b_body(acc_ref, y0_ref, dis_ref, bp_ref, wfc_ref, bfc_ref, out_ref):
    accsum = acc_ref[0] + acc_ref[1]
    dis = dis_ref[...]
    pre = y0_ref[...] - accsum * dis[:, None] + bp_ref[...]
    h = jnp.maximum(pre, 0.0)
    logits = jnp.dot(h, wfc_ref[...], preferred_element_type=jnp.float32)
    logits = logits + bfc_ref[...]
    lane = lax.broadcasted_iota(jnp.int32, logits.shape, 1)
    masked = jnp.where(lane < HID, logits, -jnp.inf)
    m = jnp.max(masked, axis=1, keepdims=True)
    s = jnp.sum(jnp.exp(masked - m), axis=1, keepdims=True)
    out_ref[...] = logits - m - jnp.log(s)


def _tc_b(acc, y0, dis, bp, wfcp, bfcp):
    return pl.pallas_call(
        _tc_b_body,
        out_shape=jax.ShapeDtypeStruct((NPAD, FP), jnp.float32),
    )(acc, y0, dis, bp, wfcp, bfcp)


@jax.jit
def kernel(x, edge_index, W0, W1, b, Wfc, bfc):
    row = edge_index[0]
    col = edge_index[1]
    pad = jnp.full((EPAD - N_EDGES,), N_NODES, jnp.int32)
    rowp = jnp.concatenate([row, pad])
    colp = jnp.concatenate([col, pad])
    row2 = rowp.reshape(NW, GPT, 16)
    row3 = rowp.reshape(NW, CPT, CHUNK)
    col3 = colp.reshape(NW, CPT, CHUNK)
    xp = jnp.pad(x, ((0, NPAD - N_NODES), (0, 0)))
    w0p = jnp.pad(W0, ((0, 0), (0, FP - HID)))
    w1p = jnp.pad(W1, ((0, 0), (0, FP - HID)))
    bp = jnp.pad(b, (0, FP - HID)).reshape(1, FP)
    wfcp = jnp.pad(Wfc, ((0, FP - HID), (0, FP - HID)))
    bfcp = jnp.pad(bfc, (0, FP - HID)).reshape(1, FP)
    zb = jnp.zeros((STRIPE, FP), jnp.float32)

    degs = _deg_kernel()(row2, zb.reshape(NPAD))
    y0, z, dis = _tc_a(xp, w0p, w1p, degs)
    acc = _msg_kernel()(row3, col3, z, zb)
    res = _tc_b(acc, y0, dis, bp, wfcp, bfcp)
    return res[:N_NODES, :HID]


# speedup vs baseline: 1.4009x; 1.4009x over previous
import functools

import jax
import jax.numpy as jnp
from jax import lax
from jax.experimental import pallas as pl
from jax.experimental.pallas import tpu as pltpu
from jax.experimental.pallas import tpu_sc as plsc

N_NODES = 10000
N_EDGES = 320000
F_IN = 128
HID = 10
FP = 16

NW = 32
CHUNK = 128
CPT = 80
EPT = CPT * CHUNK
EPAD = NW * EPT
NPAD = 10112
GPT = EPT // 16
STRIPE = NPAD // 16
NBUF = 8

@functools.cache
def _sc_mesh():
    return plsc.VectorSubcoreMesh(
        core_axis_name="c", subcore_axis_name="s", num_cores=2, num_subcores=16)


def _deg_body(row_hbm, zb1_hbm, deg_out, row_v, deg_v):
    cid = lax.axis_index("c")
    sid = lax.axis_index("s")
    wid = sid * 2 + cid
    pltpu.sync_copy(row_hbm.at[wid], row_v)
    pltpu.sync_copy(zb1_hbm, deg_v)
    ones16 = jnp.ones((16,), jnp.float32)

    def add_body(j, carry):
        plsc.addupdate_scatter(deg_v, [row_v[j]], ones16)
        return carry

    lax.fori_loop(0, GPT, add_body, 0, unroll=8)
    pltpu.sync_copy(deg_v, deg_out.at[wid])


@functools.cache
def _deg_kernel():
    return pl.kernel(
        _deg_body,
        out_type=jax.ShapeDtypeStruct((NW, NPAD), jnp.float32),
        mesh=_sc_mesh(),
        scratch_types=[
            pltpu.VMEM((GPT, 16), jnp.int32),
            pltpu.VMEM((NPAD,), jnp.float32),
        ],
        compiler_params=pltpu.CompilerParams(needs_layout_passes=False),
    )


def _msg_body(row_hbm, col_hbm, z_hbm, zb_hbm, acc_out,
              row_v, col_v, *rest):
    bufs = rest[:NBUF]
    acc_s = rest[NBUF]
    gsems = rest[NBUF + 1:2 * NBUF + 1]
    ssems = rest[2 * NBUF + 1:]
    cid = lax.axis_index("c")
    sid = lax.axis_index("s")
    wid = sid * 2 + cid
    pltpu.sync_copy(row_hbm.at[wid], row_v)
    pltpu.sync_copy(col_hbm.at[wid], col_v)
    pltpu.sync_copy(zb_hbm, acc_s.at[pl.ds(sid * STRIPE, STRIPE)])
    plsc.subcore_barrier()

    for i in range(NBUF):
        pltpu.async_copy(z_hbm.at[row_v.at[i]], bufs[i], gsems[i])

    def body(k, carry):
        j0 = NBUF * k
        for i in range(NBUF):
            pltpu.make_async_copy(
                z_hbm.at[row_v.at[j0 + i]], bufs[i], gsems[i]).wait()
            pltpu.async_copy(
                bufs[i], acc_s.at[col_v.at[j0 + i]], ssems[i], add=True)
        for i in range(NBUF):
            pltpu.make_async_copy(
                bufs[i], acc_s.at[col_v.at[j0 + i]], ssems[i]).wait()
            pltpu.async_copy(
                z_hbm.at[row_v.at[j0 + NBUF + i]], bufs[i], gsems[i])
        return carry

    lax.fori_loop(0, CPT // NBUF - 1, body, 0)
    j0 = (CPT // NBUF - 1) * NBUF
    for i in range(NBUF):
        pltpu.make_async_copy(
            z_hbm.at[row_v.at[j0 + i]], bufs[i], gsems[i]).wait()
        pltpu.async_copy(
            bufs[i], acc_s.at[col_v.at[j0 + i]], ssems[i], add=True)
    for i in range(NBUF):
        pltpu.make_async_copy(
            bufs[i], acc_s.at[col_v.at[j0 + i]], ssems[i]).wait()
    for j in range(NBUF * (CPT // NBUF), CPT):
        i = j % NBUF
        pltpu.sync_copy(z_hbm.at[row_v.at[j]], bufs[i])
        pltpu.sync_copy(bufs[i], acc_s.at[col_v.at[j]], add=True)
    plsc.subcore_barrier()
    pltpu.sync_copy(acc_s.at[pl.ds(sid * STRIPE, STRIPE)],
                    acc_out.at[cid, pl.ds(sid * STRIPE, STRIPE)])


@functools.cache
def _msg_kernel():
    return pl.kernel(
        _msg_body,
        out_type=jax.ShapeDtypeStruct((2, NPAD, FP), jnp.float32),
        mesh=_sc_mesh(),
        scratch_types=[
            pltpu.VMEM((CPT, CHUNK), jnp.int32),
            pltpu.VMEM((CPT, CHUNK), jnp.int32),
        ] + [pltpu.VMEM((CHUNK, FP), jnp.float32)] * NBUF + [
            pltpu.VMEM_SHARED((NPAD, FP), jnp.float32),
        ] + [pltpu.SemaphoreType.DMA] * (2 * NBUF),
        compiler_params=pltpu.CompilerParams(
            needs_layout_passes=False, use_tc_tiling_on_sc=False),
    )


def _tc_a_body(xp_ref, w0_ref, w1_ref, degs_ref, y0_ref, z_ref, dis_ref):
    deg = jnp.sum(degs_ref[...], axis=0)
    dis = jnp.where(deg > 0, lax.rsqrt(deg), 0.0)
    x = xp_ref[...]
    y0_ref[...] = jnp.dot(x, w0_ref[...], preferred_element_type=jnp.float32)
    y1 = jnp.dot(x, w1_ref[...], preferred_element_type=jnp.float32)
    z_ref[...] = y1 * dis[:, None]
    dis_ref[...] = dis


def _tc_a(xp, w0p, w1p, degs):
    return pl.pallas_call(
        _tc_a_body,
        out_shape=(
            jax.ShapeDtypeStruct((NPAD, FP), jnp.float32),
            jax.ShapeDtypeStruct((NPAD, FP), jnp.float32),
            jax.ShapeDtypeStruct((NPAD,), jnp.float32),
        ),
    )(xp, w0p, w1p, degs)


def _tc_b_body(acc_ref, y0_ref, dis_ref, bp_ref, wfc_ref, bfc_ref, out_ref):
    accsum = acc_ref[0] + acc_ref[1]
    dis = dis_ref[...]
    pre = y0_ref[...] - accsum * dis[:, None] + bp_ref[...]
    h = jnp.maximum(pre, 0.0)
    logits = jnp.dot(h, wfc_ref[...], preferred_element_type=jnp.float32)
    logits = logits + bfc_ref[...]
    lane = lax.broadcasted_iota(jnp.int32, logits.shape, 1)
    masked = jnp.where(lane < HID, logits, -jnp.inf)
    m = jnp.max(masked, axis=1, keepdims=True)
    s = jnp.sum(jnp.exp(masked - m), axis=1, keepdims=True)
    out_ref[...] = logits - m - jnp.log(s)


def _tc_b(acc, y0, dis, bp, wfcp, bfcp):
    return pl.pallas_call(
        _tc_b_body,
        out_shape=jax.ShapeDtypeStruct((NPAD, FP), jnp.float32),
    )(acc, y0, dis, bp, wfcp, bfcp)


@jax.jit
def kernel(x, edge_index, W0, W1, b, Wfc, bfc):
    row = edge_index[0]
    col = edge_index[1]
    pad = N_NODES + jnp.arange(EPAD - N_EDGES, dtype=jnp.int32) % (
        NPAD - N_NODES)
    rowp = jnp.concatenate([row, pad])
    colp = jnp.concatenate([col, pad])
    row2 = rowp.reshape(NW, GPT, 16)
    row3 = rowp.reshape(NW, CPT, CHUNK)
    col3 = colp.reshape(NW, CPT, CHUNK)
    xp = jnp.pad(x, ((0, NPAD - N_NODES), (0, 0)))
    w0p = jnp.pad(W0, ((0, 0), (0, FP - HID)))
    w1p = jnp.pad(W1, ((0, 0), (0, FP - HID)))
    bp = jnp.pad(b, (0, FP - HID)).reshape(1, FP)
    wfcp = jnp.pad(Wfc, ((0, FP - HID), (0, FP - HID)))
    bfcp = jnp.pad(bfc, (0, FP - HID)).reshape(1, FP)
    zb = jnp.zeros((STRIPE, FP), jnp.float32)

    degs = _deg_kernel()(row2, zb.reshape(NPAD))
    y0, z, dis = _tc_a(xp, w0p, w1p, degs)
    acc = _msg_kernel()(row3, col3, z, zb)
    res = _tc_b(acc, y0, dis, bp, wfcp, bfcp)
    return res[:N_NODES, :HID]


# speedup vs baseline: 1.4346x; 1.0241x over previous
import functools

import jax
import jax.numpy as jnp
from jax import lax
from jax.experimental import pallas as pl
from jax.experimental.pallas import tpu as pltpu
from jax.experimental.pallas import tpu_sc as plsc

N_NODES = 10000
N_EDGES = 320000
F_IN = 128
HID = 10
FP = 16

NW = 32
CHUNK = 128
CPT = 80
EPT = CPT * CHUNK
EPAD = NW * EPT
NPAD = 10112
GPT = EPT // 16
STRIPE = NPAD // 16
NBUF = 8

@functools.cache
def _sc_mesh():
    return plsc.VectorSubcoreMesh(
        core_axis_name="c", subcore_axis_name="s", num_cores=2, num_subcores=16)


def _deg_body(row_hbm, zb1_hbm, deg_out, row_v, deg_v):
    cid = lax.axis_index("c")
    sid = lax.axis_index("s")
    wid = sid * 2 + cid
    pltpu.sync_copy(row_hbm.at[wid], row_v)
    pltpu.sync_copy(zb1_hbm, deg_v)
    ones16 = jnp.ones((16,), jnp.float32)

    def add_body(j, carry):
        plsc.addupdate_scatter(deg_v, [row_v[j]], ones16)
        return carry

    lax.fori_loop(0, GPT, add_body, 0, unroll=8)
    pltpu.sync_copy(deg_v, deg_out.at[wid])


@functools.cache
def _deg_kernel():
    return pl.kernel(
        _deg_body,
        out_type=jax.ShapeDtypeStruct((NW, NPAD), jnp.float32),
        mesh=_sc_mesh(),
        scratch_types=[
            pltpu.VMEM((GPT, 16), jnp.int32),
            pltpu.VMEM((NPAD,), jnp.float32),
        ],
        compiler_params=pltpu.CompilerParams(needs_layout_passes=False),
    )


def _msg_body(row_hbm, col_hbm, z_hbm, zb_hbm, acc_out,
              row_v, col_v, *rest):
    bufs = rest[:NBUF]
    acc_s = rest[NBUF]
    gsems = rest[NBUF + 1:2 * NBUF + 1]
    ssems = rest[2 * NBUF + 1:]
    cid = lax.axis_index("c")
    sid = lax.axis_index("s")
    wid = sid * 2 + cid
    pltpu.sync_copy(row_hbm.at[wid], row_v)
    pltpu.sync_copy(col_hbm.at[wid], col_v)
    pltpu.sync_copy(zb_hbm, acc_s.at[pl.ds(sid * STRIPE, STRIPE)])
    plsc.subcore_barrier()

    for i in range(NBUF):
        pltpu.async_copy(z_hbm.at[row_v.at[i]], bufs[i], gsems[i])

    def body(k, carry):
        j0 = NBUF * k
        for i in range(NBUF):
            pltpu.make_async_copy(
                z_hbm.at[row_v.at[j0 + i]], bufs[i], gsems[i]).wait()
            pltpu.async_copy(
                bufs[i], acc_s.at[col_v.at[j0 + i]], ssems[i], add=True)
        for i in range(NBUF):
            pltpu.make_async_copy(
                bufs[i], acc_s.at[col_v.at[j0 + i]], ssems[i]).wait()
            pltpu.async_copy(
                z_hbm.at[row_v.at[j0 + NBUF + i]], bufs[i], gsems[i])
        return carry

    lax.fori_loop(0, CPT // NBUF - 1, body, 0)
    j0 = (CPT // NBUF - 1) * NBUF
    for i in range(NBUF):
        pltpu.make_async_copy(
            z_hbm.at[row_v.at[j0 + i]], bufs[i], gsems[i]).wait()
        pltpu.async_copy(
            bufs[i], acc_s.at[col_v.at[j0 + i]], ssems[i], add=True)
    for i in range(NBUF):
        pltpu.make_async_copy(
            bufs[i], acc_s.at[col_v.at[j0 + i]], ssems[i]).wait()
    for j in range(NBUF * (CPT // NBUF), CPT):
        i = j % NBUF
        pltpu.sync_copy(z_hbm.at[row_v.at[j]], bufs[i])
        pltpu.sync_copy(bufs[i], acc_s.at[col_v.at[j]], add=True)
    plsc.subcore_barrier()
    pltpu.sync_copy(acc_s.at[pl.ds(sid * STRIPE, STRIPE)],
                    acc_out.at[cid, pl.ds(sid * STRIPE, STRIPE)])


@functools.cache
def _msg_kernel():
    return pl.kernel(
        _msg_body,
        out_type=jax.ShapeDtypeStruct((2, NPAD, FP), jnp.float32),
        mesh=_sc_mesh(),
        scratch_types=[
            pltpu.VMEM((CPT, CHUNK), jnp.int32),
            pltpu.VMEM((CPT, CHUNK), jnp.int32),
        ] + [pltpu.VMEM((CHUNK, FP), jnp.float32)] * NBUF + [
            pltpu.VMEM_SHARED((NPAD, FP), jnp.float32),
        ] + [pltpu.SemaphoreType.DMA] * (2 * NBUF),
        compiler_params=pltpu.CompilerParams(
            needs_layout_passes=False, use_tc_tiling_on_sc=False),
    )


def _tc_a_body(x_ref, w0_ref, w1_ref, degs_ref, y0_ref, z_ref, dis_ref):
    deg = jnp.sum(degs_ref[...], axis=0)
    dis = jnp.where(deg > 0, lax.rsqrt(deg), 0.0)
    x = x_ref[...]
    y0_ref[...] = jnp.dot(x, w0_ref[...], preferred_element_type=jnp.float32)
    y1 = jnp.dot(x, w1_ref[...], preferred_element_type=jnp.float32)
    z_ref[...] = jnp.concatenate(
        [y1 * dis[:N_NODES, None],
         jnp.zeros((NPAD - N_NODES, FP), jnp.float32)], axis=0)
    dis_ref[...] = dis


def _tc_a(x, w0p, w1p, degs):
    return pl.pallas_call(
        _tc_a_body,
        out_shape=(
            jax.ShapeDtypeStruct((N_NODES, FP), jnp.float32),
            jax.ShapeDtypeStruct((NPAD, FP), jnp.float32),
            jax.ShapeDtypeStruct((NPAD,), jnp.float32),
        ),
    )(x, w0p, w1p, degs)


def _tc_b_body(acc_ref, y0_ref, dis_ref, bp_ref, wfc_ref, bfc_ref, out_ref):
    a = acc_ref[...]
    accsum = a[0, :N_NODES] + a[1, :N_NODES]
    dis = dis_ref[...]
    pre = y0_ref[...] - accsum * dis[:N_NODES, None] + bp_ref[...]
    h = jnp.maximum(pre, 0.0)
    logits = jnp.dot(h, wfc_ref[...], preferred_element_type=jnp.float32)
    logits = logits + bfc_ref[...]
    lane = lax.broadcasted_iota(jnp.int32, logits.shape, 1)
    masked = jnp.where(lane < HID, logits, -jnp.inf)
    m = jnp.max(masked, axis=1, keepdims=True)
    s = jnp.sum(jnp.exp(masked - m), axis=1, keepdims=True)
    out_ref[...] = logits - m - jnp.log(s)


def _tc_b(acc, y0, dis, bp, wfcp, bfcp):
    return pl.pallas_call(
        _tc_b_body,
        out_shape=jax.ShapeDtypeStruct((N_NODES, FP), jnp.float32),
    )(acc, y0, dis, bp, wfcp, bfcp)


@jax.jit
def kernel(x, edge_index, W0, W1, b, Wfc, bfc):
    row = edge_index[0]
    col = edge_index[1]
    pad = N_NODES + jnp.arange(EPAD - N_EDGES, dtype=jnp.int32) % (
        NPAD - N_NODES)
    rowp = jnp.concatenate([row, pad])
    colp = jnp.concatenate([col, pad])
    row2 = rowp.reshape(NW, GPT, 16)
    row3 = rowp.reshape(NW, CPT, CHUNK)
    col3 = colp.reshape(NW, CPT, CHUNK)
    w0p = jnp.pad(W0, ((0, 0), (0, FP - HID)))
    w1p = jnp.pad(W1, ((0, 0), (0, FP - HID)))
    bp = jnp.pad(b, (0, FP - HID)).reshape(1, FP)
    wfcp = jnp.pad(Wfc, ((0, FP - HID), (0, FP - HID)))
    bfcp = jnp.pad(bfc, (0, FP - HID)).reshape(1, FP)
    zb = jnp.zeros((STRIPE, FP), jnp.float32)

    degs = _deg_kernel()(row2, zb.reshape(NPAD))
    y0, z, dis = _tc_a(x, w0p, w1p, degs)
    acc = _msg_kernel()(row3, col3, z, zb)
    res = _tc_b(acc, y0, dis, bp, wfcp, bfcp)
    return res[:, :HID]
